# baseline (speedup 1.0000x reference)
import functools

import jax
import jax.numpy as jnp
from jax.experimental import pallas as pl
from jax.experimental.pallas import tpu as pltpu

B, D, H, W = 8, 128, 128, 128
HW = H * W
MI = 23
WB = 512
NJ = HW // WB


def _tc_body(enc_ref, msk_ref, out_ref, acc_ref):
    j = pl.program_id(1)

    @pl.when(j == 0)
    def _init():
        acc_ref[...] = jnp.full_like(acc_ref, -jnp.inf)

    enc = enc_ref[0]
    neg = jnp.float32(-jnp.inf)
    for i in range(MI):
        m = msk_ref[0, i, :] > 0
        mb = jnp.broadcast_to(m[None, :], (D, WB))
        acc_ref[i] = jnp.maximum(acc_ref[i], jnp.where(mb, enc, neg))

    @pl.when(j == NJ - 1)
    def _finish():
        out_ref[0] = jnp.max(acc_ref[...], axis=-1)


@jax.jit
def kernel(encoded, masks):
    enc = encoded.reshape(B, D, HW)
    msk = masks[:, 1:, :, :].reshape(B, MI, HW)
    out = pl.pallas_call(
        _tc_body,
        grid=(B, NJ),
        in_specs=[
            pl.BlockSpec((1, D, WB), lambda b, j: (b, 0, j)),
            pl.BlockSpec((1, MI, WB), lambda b, j: (b, 0, j)),
        ],
        out_specs=pl.BlockSpec((1, MI, D), lambda b, j: (b, 0, 0)),
        out_shape=jax.ShapeDtypeStruct((B, MI, D), jnp.float32),
        scratch_shapes=[pltpu.VMEM((MI, D, WB), jnp.float32)],
        compiler_params=pltpu.CompilerParams(
            dimension_semantics=("arbitrary", "arbitrary"),
        ),
    )(enc, msk)
    return jnp.transpose(out, (0, 2, 1))[:, :, :, None]


# speedup vs baseline: 1.9394x; 1.9394x over previous
import functools

import jax
import jax.numpy as jnp
from jax.experimental import pallas as pl
from jax.experimental.pallas import tpu as pltpu

B, D, H, W = 8, 128, 128, 128
HW = H * W
MI = 23
WB = 512
NJ = HW // WB


def _tc_body(enc_ref, msk_ref, out_ref, acc_ref):
    j = pl.program_id(1)

    @pl.when(j == 0)
    def _init():
        acc_ref[...] = jnp.full_like(acc_ref, -jnp.inf)

    enc = enc_ref[0].astype(jnp.bfloat16)
    bias = jnp.where(msk_ref[0] > 0, jnp.float32(0), jnp.float32(-jnp.inf))
    bias = bias.astype(jnp.bfloat16)
    for i in range(MI):
        bi = jnp.broadcast_to(bias[i][None, :], (D, WB))
        masked = enc + bi
        f = jnp.maximum(masked[:, :WB // 2], masked[:, WB // 2:])
        f = jnp.maximum(f[:, :WB // 4], f[:, WB // 4:])
        acc_ref[i] = jnp.maximum(acc_ref[i], f)

    @pl.when(j == NJ - 1)
    def _finish():
        out_ref[0] = jnp.max(acc_ref[...], axis=-1).astype(jnp.float32)


@jax.jit
def kernel(encoded, masks):
    enc = encoded.reshape(B, D, HW)
    msk = masks[:, 1:, :, :].reshape(B, MI, HW)
    out = pl.pallas_call(
        _tc_body,
        grid=(B, NJ),
        in_specs=[
            pl.BlockSpec((1, D, WB), lambda b, j: (b, 0, j)),
            pl.BlockSpec((1, MI, WB), lambda b, j: (b, 0, j)),
        ],
        out_specs=pl.BlockSpec((1, MI, D), lambda b, j: (b, 0, 0)),
        out_shape=jax.ShapeDtypeStruct((B, MI, D), jnp.float32),
        scratch_shapes=[pltpu.VMEM((MI, D, WB // 4), jnp.bfloat16)],
        compiler_params=pltpu.CompilerParams(
            dimension_semantics=("arbitrary", "arbitrary"),
        ),
    )(enc, msk)
    return jnp.transpose(out, (0, 2, 1))[:, :, :, None]


# speedup vs baseline: 2.2939x; 1.1828x over previous
import functools

import jax
import jax.numpy as jnp
from jax.experimental import pallas as pl
from jax.experimental.pallas import tpu as pltpu

B, D, H, W = 8, 128, 128, 128
HW = H * W
MI = 23
WB = 1024
NJ = HW // WB
WBP = 4096
NJP = HW // WBP


def _bias_body(msk_ref, bias_ref):
    m = msk_ref[0]
    bias = jnp.where(m > 0, jnp.float32(0), jnp.float32(-jnp.inf))
    bias_ref[0] = bias.astype(jnp.bfloat16)


def _tc_body(enc_ref, bias_ref, out_ref, acc_ref):
    j = pl.program_id(1)

    @pl.when(j == 0)
    def _init():
        acc_ref[...] = jnp.full_like(acc_ref, -jnp.inf)

    enc = enc_ref[0].astype(jnp.bfloat16)
    for i in range(MI):
        bi = jnp.broadcast_to(bias_ref[0, i][None, :], (D, WB))
        masked = enc + bi
        f = jnp.maximum(masked[:, :WB // 2], masked[:, WB // 2:])
        f = jnp.maximum(f[:, :WB // 4], f[:, WB // 4:])
        f = jnp.maximum(f[:, :WB // 8], f[:, WB // 8:])
        acc_ref[i] = jnp.maximum(acc_ref[i], f)

    @pl.when(j == NJ - 1)
    def _finish():
        out_ref[0] = jnp.max(acc_ref[...], axis=-1).astype(jnp.float32)


@jax.jit
def kernel(encoded, masks):
    enc = encoded.reshape(B, D, HW)
    msk = masks[:, 1:, :, :].reshape(B, MI, HW)
    bias = pl.pallas_call(
        _bias_body,
        grid=(B, NJP),
        in_specs=[pl.BlockSpec((1, MI, WBP), lambda b, j: (b, 0, j))],
        out_specs=pl.BlockSpec((1, MI, WBP), lambda b, j: (b, 0, j)),
        out_shape=jax.ShapeDtypeStruct((B, MI, HW), jnp.bfloat16),
    )(msk)
    out = pl.pallas_call(
        _tc_body,
        grid=(B, NJ),
        in_specs=[
            pl.BlockSpec((1, D, WB), lambda b, j: (b, 0, j)),
            pl.BlockSpec((1, MI, WB), lambda b, j: (b, 0, j)),
        ],
        out_specs=pl.BlockSpec((1, MI, D), lambda b, j: (b, 0, 0)),
        out_shape=jax.ShapeDtypeStruct((B, MI, D), jnp.float32),
        scratch_shapes=[pltpu.VMEM((MI, D, 128), jnp.bfloat16)],
        compiler_params=pltpu.CompilerParams(
            dimension_semantics=("arbitrary", "arbitrary"),
        ),
    )(enc, bias)
    return jnp.transpose(out, (0, 2, 1))[:, :, :, None]
